# baseline (speedup 1.0000x reference)
import functools

import jax
import jax.numpy as jnp
from jax import lax
from jax.experimental import pallas as pl
from jax.experimental.pallas import tpu as pltpu
from jax.experimental.pallas import tpu_sc as plsc

N_NODES = 50000
N_GRAPHS = 16

E_PAD = 819200
N_PAD = 50176
BE = 2048
BN = 1024
DUMP = N_NODES

SC_K = 1600
EPW_G = E_PAD // 32
EPT_S = E_PAD // 16
NPT = N_PAD // 16
_SC_MESH = dict(core_axis_name="c", subcore_axis_name="s")


def _ured_body(u_ref, wu_ref, bu_ref, out_ref):
    out_ref[...] = (
        jnp.dot(u_ref[...], wu_ref[...], preferred_element_type=jnp.float32)
        + bu_ref[...]
    )


def _edge_body(xr_ref, xc_ref, ea_ref, ured_ref, w0r_ref, w0c_ref, w0u_ref,
               w0e_ref, be0_ref, we1_ref, be1_ref, we2_ref, be2_ref, w1c_ref,
               w1e_ref, bn10_ref, wn11_ref, bn11_ref,
               m0_ref, m1_ref, m2_ref, m3_ref):
    f32 = jnp.float32
    bf16 = jnp.bfloat16
    xr = xr_ref[...]
    xc = xc_ref[...]
    xrb = xr.astype(bf16)
    xcb = xc.astype(bf16)
    b = xr[:, 9:10]
    iota = lax.broadcasted_iota(jnp.int32, (1, N_GRAPHS), 1).astype(f32)
    oh = (b == iota).astype(bf16)
    ub = jnp.dot(oh, ured_ref[...].astype(bf16),
                 preferred_element_type=f32).astype(bf16)
    e0 = (jnp.dot(xrb, w0r_ref[...].astype(bf16), preferred_element_type=f32)
          + jnp.dot(xcb, w0c_ref[...].astype(bf16), preferred_element_type=f32)
          + jnp.dot(ub, w0u_ref[...].astype(bf16), preferred_element_type=f32)
          + ea_ref[...] * w0e_ref[...]
          + be0_ref[...])
    h = jnp.maximum(e0, 0.0).astype(bf16)
    h = jnp.maximum(jnp.dot(h, we1_ref[...].astype(bf16),
                            preferred_element_type=f32)
                    + be1_ref[...], 0.0).astype(bf16)
    eo = jnp.dot(h, we2_ref[...].astype(bf16),
                 preferred_element_type=f32) + be2_ref[...]
    m = jnp.maximum(jnp.dot(xcb, w1c_ref[...].astype(bf16),
                            preferred_element_type=f32)
                    + jnp.dot(eo.astype(bf16), w1e_ref[...].astype(bf16),
                              preferred_element_type=f32)
                    + bn10_ref[...], 0.0).astype(bf16)
    m = jnp.maximum(jnp.dot(m, wn11_ref[...].astype(bf16),
                            preferred_element_type=f32)
                    + bn11_ref[...], 0.0)
    m0_ref[...] = m[:, 0:16]
    m1_ref[...] = m[:, 16:32]
    m2_ref[...] = m[:, 32:48]
    m3_ref[...] = m[:, 48:64]


def _node_body(x_ref, s0_ref, s1_ref, s2_ref, s3_ref, cnt_ref, batch_ref,
               ured_ref, w2x_ref, w2a_ref, w2b_ref, w2c_ref, w2d_ref,
               w2u_ref, bn20_ref, wn21_ref, bn21_ref, out_ref):
    f32 = jnp.float32
    inv = 1.0 / jnp.maximum(cnt_ref[...], 1.0)
    b = batch_ref[...]
    oh = (b == lax.broadcasted_iota(jnp.int32, (1, N_GRAPHS), 1)).astype(f32)
    ub = jnp.dot(oh, ured_ref[...], preferred_element_type=f32)
    h2 = (jnp.dot(x_ref[...], w2x_ref[...], preferred_element_type=f32)
          + jnp.dot(s0_ref[...] * inv, w2a_ref[...], preferred_element_type=f32)
          + jnp.dot(s1_ref[...] * inv, w2b_ref[...], preferred_element_type=f32)
          + jnp.dot(s2_ref[...] * inv, w2c_ref[...], preferred_element_type=f32)
          + jnp.dot(s3_ref[...] * inv, w2d_ref[...], preferred_element_type=f32)
          + jnp.dot(ub, w2u_ref[...], preferred_element_type=f32)
          + bn20_ref[...])
    h2 = jnp.maximum(h2, 0.0)
    out_ref[...] = (jnp.dot(h2, wn21_ref[...], preferred_element_type=f32)
                    + bn21_ref[...])


def _full(shape):
    return pl.BlockSpec(shape, lambda i: (0,) * len(shape))


def _gather_body(x16_hbm, rowp_hbm, colp_hbm, xr_hbm, xcp_hbm,
                 row_v, col_v, xr_v, xc_v, sem1, sem2):
    c = lax.axis_index("c")
    s = lax.axis_index("s")
    wid = s * 2 + c
    base_w = wid * EPW_G

    @pl.loop(0, EPW_G // SC_K)
    def _chunk(it):
        eb = base_w + it * SC_K
        pltpu.sync_copy(rowp_hbm.at[pl.ds(eb, SC_K)], row_v)
        pltpu.sync_copy(colp_hbm.at[pl.ds(eb, SC_K)], col_v)
        cp1 = pltpu.async_copy(x16_hbm.at[row_v], xr_v, sem1)
        cp2 = pltpu.async_copy(x16_hbm.at[col_v], xc_v, sem2)
        cp1.wait()
        cp2.wait()
        pltpu.sync_copy(xr_v, xr_hbm.at[pl.ds(eb, SC_K)])
        pltpu.sync_copy(xc_v, xcp_hbm.at[pl.ds(eb, SC_K)])


def _sc_gather(x16, rowp, colp):
    f32 = jnp.float32
    return pl.kernel(
        _gather_body,
        out_type=[
            jax.ShapeDtypeStruct((E_PAD, 16), f32),
            jax.ShapeDtypeStruct((E_PAD, 16), f32),
        ],
        mesh=plsc.VectorSubcoreMesh(**_SC_MESH),
        scratch_types=[
            pltpu.VMEM((SC_K,), jnp.int32),
            pltpu.VMEM((SC_K,), jnp.int32),
            pltpu.VMEM((SC_K, 16), f32),
            pltpu.VMEM((SC_K, 16), f32),
            pltpu.SemaphoreType.DMA,
            pltpu.SemaphoreType.DMA,
        ],
        compiler_params=pltpu.CompilerParams(use_tc_tiling_on_sc=False),
    )(x16, rowp, colp)


def _scatter_body(m0_hbm, m1_hbm, m2_hbm, m3_hbm, rowp_hbm, z16_hbm, zcol_hbm,
                  ones_hbm, s0_hbm, s1_hbm, s2_hbm, s3_hbm, cnt_hbm,
                  row_v, m_v, ones_v, acc_sh, cnt_sh):
    c = lax.axis_index("c")
    s = lax.axis_index("s")
    r0 = s * NPT
    base_t = s * EPT_S

    for p in range(2):
        pltpu.sync_copy(z16_hbm, acc_sh.at[pl.ds(r0, NPT)])
        if p == 0:
            @pl.when(c == 0)
            def _():
                pltpu.sync_copy(zcol_hbm, cnt_sh.at[pl.ds(r0, NPT)])
                pltpu.sync_copy(ones_hbm, ones_v)

        plsc.subcore_barrier()

        @pl.loop(0, EPT_S // SC_K)
        def _chunk(it):
            eb = base_t + it * SC_K
            pltpu.sync_copy(rowp_hbm.at[pl.ds(eb, SC_K)], row_v)

            @pl.when(c == 0)
            def _():
                pltpu.sync_copy((m0_hbm, m1_hbm)[p].at[pl.ds(eb, SC_K)], m_v)

            @pl.when(c == 1)
            def _():
                pltpu.sync_copy((m2_hbm, m3_hbm)[p].at[pl.ds(eb, SC_K)], m_v)

            pltpu.sync_copy(m_v, acc_sh.at[row_v], add=True)
            if p == 0:
                @pl.when(c == 0)
                def _():
                    pltpu.sync_copy(ones_v, cnt_sh.at[row_v], add=True)

        plsc.subcore_barrier()

        @pl.when(c == 0)
        def _():
            pltpu.sync_copy(acc_sh.at[pl.ds(r0, NPT)],
                            (s0_hbm, s1_hbm)[p].at[pl.ds(r0, NPT)])
            if p == 0:
                pltpu.sync_copy(cnt_sh.at[pl.ds(r0, NPT)],
                                cnt_hbm.at[pl.ds(r0, NPT)])

        @pl.when(c == 1)
        def _():
            pltpu.sync_copy(acc_sh.at[pl.ds(r0, NPT)],
                            (s2_hbm, s3_hbm)[p].at[pl.ds(r0, NPT)])


def _sc_scatter(m0, m1, m2, m3, rowp):
    f32 = jnp.float32
    z16 = jnp.zeros((NPT, 16), f32)
    zcol = jnp.zeros((NPT,), f32)
    ones = jnp.ones((SC_K,), f32)
    return pl.kernel(
        _scatter_body,
        out_type=[
            jax.ShapeDtypeStruct((N_PAD, 16), f32),
            jax.ShapeDtypeStruct((N_PAD, 16), f32),
            jax.ShapeDtypeStruct((N_PAD, 16), f32),
            jax.ShapeDtypeStruct((N_PAD, 16), f32),
            jax.ShapeDtypeStruct((N_PAD,), f32),
        ],
        mesh=plsc.VectorSubcoreMesh(**_SC_MESH),
        scratch_types=[
            pltpu.VMEM((SC_K,), jnp.int32),
            pltpu.VMEM((SC_K, 16), f32),
            pltpu.VMEM((SC_K,), f32),
            pltpu.VMEM_SHARED((N_PAD, 16), f32),
            pltpu.VMEM_SHARED((N_PAD,), f32),
        ],
        compiler_params=pltpu.CompilerParams(use_tc_tiling_on_sc=False),
    )(m0, m1, m2, m3, rowp, z16, zcol, ones)


def kernel(x, edge_index, edge_attr, u, batch, Wu, bu, We0, be0, We1, be1,
           We2, be2, Wn10, bn10, Wn11, bn11, Wn20, bn20, Wn21, bn21):
    f32 = jnp.float32
    row = edge_index[0]
    col = edge_index[1]
    ne = row.shape[0]

    batchp = jnp.pad(batch, (0, N_PAD - N_NODES))
    x16 = jnp.pad(x, ((0, N_PAD - N_NODES), (0, 16 - x.shape[1])))
    x16 = x16.at[:, 9].set(batchp.astype(f32))
    rowp = jnp.concatenate([row, jnp.full((E_PAD - ne,), DUMP, jnp.int32)])
    colp = jnp.concatenate([col, jnp.zeros((E_PAD - ne,), jnp.int32)])
    eap = jnp.concatenate([edge_attr[:, 0], jnp.zeros((E_PAD - ne,), f32)])

    z64 = jnp.zeros((16, 64), f32)
    W0r = z64.at[:9].set(We0[0:9])
    W0c = z64.at[:9].set(We0[9:18])
    W0u = We0[19:51]
    w0e = We0[18:19]
    W1c = z64.at[:9].set(Wn10[0:9])
    W1e = Wn10[9:73]
    W2x = z64.at[:9].set(Wn20[0:9])
    W2a = Wn20[9:25]
    W2b = Wn20[25:41]
    W2c = Wn20[41:57]
    W2d = Wn20[57:73]
    W2u = Wn20[73:105]
    be0r = be0.reshape(1, -1)
    be1r = be1.reshape(1, -1)
    be2r = be2.reshape(1, -1)
    bn10r = bn10.reshape(1, -1)
    bn11r = bn11.reshape(1, -1)
    bn20r = bn20.reshape(1, -1)
    bn21r = bn21.reshape(1, -1)
    bur = bu.reshape(1, -1)

    u_red = pl.pallas_call(
        _ured_body,
        grid=(1,),
        in_specs=[_full((16, 4096)), _full((4096, 32)), _full((1, 32))],
        out_specs=_full((16, 32)),
        out_shape=jax.ShapeDtypeStruct((16, 32), f32),
    )(u, Wu, bur)

    xr, xc = _sc_gather(x16, rowp, colp)

    ge = E_PAD // BE
    m0, m1, m2, m3 = pl.pallas_call(
        _edge_body,
        grid=(ge,),
        in_specs=[
            pl.BlockSpec((BE, 16), lambda i: (i, 0)),
            pl.BlockSpec((BE, 16), lambda i: (i, 0)),
            pl.BlockSpec((BE, 1), lambda i: (i, 0)),
            _full((16, 32)),
            _full((16, 64)), _full((16, 64)), _full((32, 64)), _full((1, 64)),
            _full((1, 64)),
            _full((64, 64)), _full((1, 64)),
            _full((64, 64)), _full((1, 64)),
            _full((16, 64)), _full((64, 64)), _full((1, 64)),
            _full((64, 64)), _full((1, 64)),
        ],
        out_specs=[
            pl.BlockSpec((BE, 16), lambda i: (i, 0)),
            pl.BlockSpec((BE, 16), lambda i: (i, 0)),
            pl.BlockSpec((BE, 16), lambda i: (i, 0)),
            pl.BlockSpec((BE, 16), lambda i: (i, 0)),
        ],
        out_shape=[
            jax.ShapeDtypeStruct((E_PAD, 16), f32),
            jax.ShapeDtypeStruct((E_PAD, 16), f32),
            jax.ShapeDtypeStruct((E_PAD, 16), f32),
            jax.ShapeDtypeStruct((E_PAD, 16), f32),
        ],
    )(xr, xc, eap[:, None], u_red, W0r, W0c, W0u, w0e, be0r, We1, be1r,
      We2, be2r, W1c, W1e, bn10r, Wn11, bn11r)

    s0, s1, s2, s3, cnt = _sc_scatter(m0, m1, m2, m3, rowp)

    gn = N_PAD // BN
    out = pl.pallas_call(
        _node_body,
        grid=(gn,),
        in_specs=[
            pl.BlockSpec((BN, 16), lambda i: (i, 0)),
            pl.BlockSpec((BN, 16), lambda i: (i, 0)),
            pl.BlockSpec((BN, 16), lambda i: (i, 0)),
            pl.BlockSpec((BN, 16), lambda i: (i, 0)),
            pl.BlockSpec((BN, 16), lambda i: (i, 0)),
            pl.BlockSpec((BN, 1), lambda i: (i, 0)),
            pl.BlockSpec((BN, 1), lambda i: (i, 0)),
            _full((16, 32)),
            _full((16, 64)), _full((16, 64)), _full((16, 64)), _full((16, 64)),
            _full((16, 64)), _full((32, 64)),
            _full((1, 64)), _full((64, 1)), _full((1, 1)),
        ],
        out_specs=pl.BlockSpec((BN, 1), lambda i: (i, 0)),
        out_shape=jax.ShapeDtypeStruct((N_PAD, 1), f32),
    )(x16, s0, s1, s2, s3, cnt[:, None], batchp[:, None], u_red,
      W2x, W2a, W2b, W2c, W2d, W2u, bn20r, Wn21, bn21r)

    return out[:N_NODES, 0]


# speedup vs baseline: 1.0277x; 1.0277x over previous
import jax
import jax.numpy as jnp
from jax import lax
from jax.experimental import pallas as pl
from jax.experimental.pallas import tpu as pltpu
from jax.experimental.pallas import tpu_sc as plsc

N_NODES = 50000
N_GRAPHS = 16

E_PAD = 819200
N_PAD = 50176
BE = 2048
BN = 1024
DUMP = N_NODES

SC_K = 1600
NPT = N_PAD // 16
_SC_MESH = dict(core_axis_name="c", subcore_axis_name="s")
_SC_PARAMS = pltpu.CompilerParams(use_tc_tiling_on_sc=False)


def _ured_body(u_ref, wu_ref, bu_ref, out_ref):
    out_ref[...] = (
        jnp.dot(u_ref[...], wu_ref[...], preferred_element_type=jnp.float32)
        + bu_ref[...]
    )


def _edge_body(xr_ref, xc_ref, ea_ref, ured_ref, w0r_ref, w0c_ref, w0u_ref,
               w0e_ref, be0_ref, we1_ref, be1_ref, we2_ref, be2_ref, w1c_ref,
               w1e_ref, bn10_ref, wn11_ref, bn11_ref,
               m0_ref, m1_ref, m2_ref, m3_ref):
    f32 = jnp.float32
    xr = xr_ref[...]
    xc = xc_ref[...]
    b = xr[:, 9:10]
    iota = lax.broadcasted_iota(jnp.int32, (1, N_GRAPHS), 1).astype(f32)
    oh = (b == iota).astype(f32)
    ub = jnp.dot(oh, ured_ref[...], preferred_element_type=f32)
    e0 = (jnp.dot(xr, w0r_ref[...], preferred_element_type=f32)
          + jnp.dot(xc, w0c_ref[...], preferred_element_type=f32)
          + jnp.dot(ub, w0u_ref[...], preferred_element_type=f32)
          + ea_ref[...] * w0e_ref[...]
          + be0_ref[...])
    h = jnp.maximum(e0, 0.0)
    h = jnp.maximum(jnp.dot(h, we1_ref[...], preferred_element_type=f32)
                    + be1_ref[...], 0.0)
    eo = jnp.dot(h, we2_ref[...], preferred_element_type=f32) + be2_ref[...]
    m = jnp.maximum(jnp.dot(xc, w1c_ref[...], preferred_element_type=f32)
                    + jnp.dot(eo, w1e_ref[...], preferred_element_type=f32)
                    + bn10_ref[...], 0.0)
    m = jnp.maximum(jnp.dot(m, wn11_ref[...], preferred_element_type=f32)
                    + bn11_ref[...], 0.0)
    m0_ref[...] = m[:, 0:16]
    m1_ref[...] = m[:, 16:32]
    m2_ref[...] = m[:, 32:48]
    m3_ref[...] = m[:, 48:64]


def _node_body(x_ref, s0a_ref, s1a_ref, s2a_ref, s3a_ref,
               s0b_ref, s1b_ref, s2b_ref, s3b_ref, cnta_ref, cntb_ref,
               batch_ref, ured_ref, w2x_ref, w2a_ref, w2b_ref, w2c_ref,
               w2d_ref, w2u_ref, bn20_ref, wn21_ref, bn21_ref, out_ref):
    f32 = jnp.float32
    inv = 1.0 / jnp.maximum(cnta_ref[...] + cntb_ref[...], 1.0)
    b = batch_ref[...]
    oh = (b == lax.broadcasted_iota(jnp.int32, (1, N_GRAPHS), 1)).astype(f32)
    ub = jnp.dot(oh, ured_ref[...], preferred_element_type=f32)
    h2 = (jnp.dot(x_ref[...], w2x_ref[...], preferred_element_type=f32)
          + jnp.dot((s0a_ref[...] + s0b_ref[...]) * inv, w2a_ref[...],
                    preferred_element_type=f32)
          + jnp.dot((s1a_ref[...] + s1b_ref[...]) * inv, w2b_ref[...],
                    preferred_element_type=f32)
          + jnp.dot((s2a_ref[...] + s2b_ref[...]) * inv, w2c_ref[...],
                    preferred_element_type=f32)
          + jnp.dot((s3a_ref[...] + s3b_ref[...]) * inv, w2d_ref[...],
                    preferred_element_type=f32)
          + jnp.dot(ub, w2u_ref[...], preferred_element_type=f32)
          + bn20_ref[...])
    h2 = jnp.maximum(h2, 0.0)
    out_ref[...] = (jnp.dot(h2, wn21_ref[...], preferred_element_type=f32)
                    + bn21_ref[...])


def _full(shape):
    return pl.BlockSpec(shape, lambda i: (0,) * len(shape))


def _gather_body(x16_hbm, rowp_hbm, colp_hbm, xr_hbm, xcp_hbm,
                 row_v, col_v, xr_v, xc_v, sem1, sem2):
    c = lax.axis_index("c")
    s = lax.axis_index("s")
    wid = s * 2 + c
    epw = rowp_hbm.shape[0] // 32
    base_w = wid * epw

    @pl.loop(0, epw // SC_K)
    def _chunk(it):
        eb = base_w + it * SC_K
        pltpu.sync_copy(rowp_hbm.at[pl.ds(eb, SC_K)], row_v)
        pltpu.sync_copy(colp_hbm.at[pl.ds(eb, SC_K)], col_v)
        cp1 = pltpu.async_copy(x16_hbm.at[row_v], xr_v, sem1)
        cp2 = pltpu.async_copy(x16_hbm.at[col_v], xc_v, sem2)
        cp1.wait()
        cp2.wait()
        pltpu.sync_copy(xr_v, xr_hbm.at[pl.ds(eb, SC_K)])
        pltpu.sync_copy(xc_v, xcp_hbm.at[pl.ds(eb, SC_K)])


def _sc_gather(x16, rowp, colp):
    f32 = jnp.float32
    ne = rowp.shape[0]
    return pl.kernel(
        _gather_body,
        out_type=[
            jax.ShapeDtypeStruct((ne, 16), f32),
            jax.ShapeDtypeStruct((ne, 16), f32),
        ],
        mesh=plsc.VectorSubcoreMesh(**_SC_MESH),
        scratch_types=[
            pltpu.VMEM((SC_K,), jnp.int32),
            pltpu.VMEM((SC_K,), jnp.int32),
            pltpu.VMEM((SC_K, 16), f32),
            pltpu.VMEM((SC_K, 16), f32),
            pltpu.SemaphoreType.DMA,
            pltpu.SemaphoreType.DMA,
        ],
        compiler_params=_SC_PARAMS,
    )(x16, rowp, colp)


def _scatter_body(m0_hbm, m1_hbm, m2_hbm, m3_hbm, rowp_hbm, z16_hbm, zcol_hbm,
                  ones_hbm, s0_hbm, s1_hbm, s2_hbm, s3_hbm, cnt_hbm,
                  row_v, m_v, ones_v, acc_sh, cnt_sh):
    c = lax.axis_index("c")
    s = lax.axis_index("s")
    r0 = s * NPT
    ept = rowp_hbm.shape[0] // 16
    base_t = s * ept

    for p in range(2):
        pltpu.sync_copy(z16_hbm, acc_sh.at[pl.ds(r0, NPT)])
        if p == 0:
            @pl.when(c == 0)
            def _():
                pltpu.sync_copy(zcol_hbm, cnt_sh.at[pl.ds(r0, NPT)])
                pltpu.sync_copy(ones_hbm, ones_v)

        plsc.subcore_barrier()

        @pl.loop(0, ept // SC_K)
        def _chunk(it):
            eb = base_t + it * SC_K
            pltpu.sync_copy(rowp_hbm.at[pl.ds(eb, SC_K)], row_v)

            @pl.when(c == 0)
            def _():
                pltpu.sync_copy((m0_hbm, m1_hbm)[p].at[pl.ds(eb, SC_K)], m_v)

            @pl.when(c == 1)
            def _():
                pltpu.sync_copy((m2_hbm, m3_hbm)[p].at[pl.ds(eb, SC_K)], m_v)

            pltpu.sync_copy(m_v, acc_sh.at[row_v], add=True)
            if p == 0:
                @pl.when(c == 0)
                def _():
                    pltpu.sync_copy(ones_v, cnt_sh.at[row_v], add=True)

        plsc.subcore_barrier()

        @pl.when(c == 0)
        def _():
            pltpu.sync_copy(acc_sh.at[pl.ds(r0, NPT)],
                            (s0_hbm, s1_hbm)[p].at[pl.ds(r0, NPT)])
            if p == 0:
                pltpu.sync_copy(cnt_sh.at[pl.ds(r0, NPT)],
                                cnt_hbm.at[pl.ds(r0, NPT)])

        @pl.when(c == 1)
        def _():
            pltpu.sync_copy(acc_sh.at[pl.ds(r0, NPT)],
                            (s2_hbm, s3_hbm)[p].at[pl.ds(r0, NPT)])


def _sc_scatter(m0, m1, m2, m3, rowp):
    f32 = jnp.float32
    z16 = jnp.zeros((NPT, 16), f32)
    zcol = jnp.zeros((NPT,), f32)
    ones = jnp.ones((SC_K,), f32)
    return pl.kernel(
        _scatter_body,
        out_type=[
            jax.ShapeDtypeStruct((N_PAD, 16), f32),
            jax.ShapeDtypeStruct((N_PAD, 16), f32),
            jax.ShapeDtypeStruct((N_PAD, 16), f32),
            jax.ShapeDtypeStruct((N_PAD, 16), f32),
            jax.ShapeDtypeStruct((N_PAD,), f32),
        ],
        mesh=plsc.VectorSubcoreMesh(**_SC_MESH),
        scratch_types=[
            pltpu.VMEM((SC_K,), jnp.int32),
            pltpu.VMEM((SC_K, 16), f32),
            pltpu.VMEM((SC_K,), f32),
            pltpu.VMEM_SHARED((N_PAD, 16), f32),
            pltpu.VMEM_SHARED((N_PAD,), f32),
        ],
        compiler_params=_SC_PARAMS,
    )(m0, m1, m2, m3, rowp, z16, zcol, ones)


def _edge_call(xr, xc, eap, u_red, W0r, W0c, W0u, w0e, be0r, We1, be1r,
               We2, be2r, W1c, W1e, bn10r, Wn11, bn11r):
    f32 = jnp.float32
    ne = xr.shape[0]
    ge = ne // BE
    return pl.pallas_call(
        _edge_body,
        grid=(ge,),
        in_specs=[
            pl.BlockSpec((BE, 16), lambda i: (i, 0)),
            pl.BlockSpec((BE, 16), lambda i: (i, 0)),
            pl.BlockSpec((BE, 1), lambda i: (i, 0)),
            _full((16, 32)),
            _full((16, 64)), _full((16, 64)), _full((32, 64)), _full((1, 64)),
            _full((1, 64)),
            _full((64, 64)), _full((1, 64)),
            _full((64, 64)), _full((1, 64)),
            _full((16, 64)), _full((64, 64)), _full((1, 64)),
            _full((64, 64)), _full((1, 64)),
        ],
        out_specs=[
            pl.BlockSpec((BE, 16), lambda i: (i, 0)),
            pl.BlockSpec((BE, 16), lambda i: (i, 0)),
            pl.BlockSpec((BE, 16), lambda i: (i, 0)),
            pl.BlockSpec((BE, 16), lambda i: (i, 0)),
        ],
        out_shape=[
            jax.ShapeDtypeStruct((ne, 16), f32),
            jax.ShapeDtypeStruct((ne, 16), f32),
            jax.ShapeDtypeStruct((ne, 16), f32),
            jax.ShapeDtypeStruct((ne, 16), f32),
        ],
    )(xr, xc, eap[:, None], u_red, W0r, W0c, W0u, w0e, be0r, We1, be1r,
      We2, be2r, W1c, W1e, bn10r, Wn11, bn11r)


def kernel(x, edge_index, edge_attr, u, batch, Wu, bu, We0, be0, We1, be1,
           We2, be2, Wn10, bn10, Wn11, bn11, Wn20, bn20, Wn21, bn21):
    f32 = jnp.float32
    row = edge_index[0]
    col = edge_index[1]
    ne = row.shape[0]

    batchp = jnp.pad(batch, (0, N_PAD - N_NODES))
    x16 = jnp.pad(x, ((0, N_PAD - N_NODES), (0, 16 - x.shape[1])))
    x16 = x16.at[:, 9].set(batchp.astype(f32))
    rowp = jnp.concatenate([row, jnp.full((E_PAD - ne,), DUMP, jnp.int32)])
    colp = jnp.concatenate([col, jnp.zeros((E_PAD - ne,), jnp.int32)])
    eap = jnp.concatenate([edge_attr[:, 0], jnp.zeros((E_PAD - ne,), f32)])

    z64 = jnp.zeros((16, 64), f32)
    W0r = z64.at[:9].set(We0[0:9])
    W0c = z64.at[:9].set(We0[9:18])
    W0u = We0[19:51]
    w0e = We0[18:19]
    W1c = z64.at[:9].set(Wn10[0:9])
    W1e = Wn10[9:73]
    W2x = z64.at[:9].set(Wn20[0:9])
    W2a = Wn20[9:25]
    W2b = Wn20[25:41]
    W2c = Wn20[41:57]
    W2d = Wn20[57:73]
    W2u = Wn20[73:105]
    be0r = be0.reshape(1, -1)
    be1r = be1.reshape(1, -1)
    be2r = be2.reshape(1, -1)
    bn10r = bn10.reshape(1, -1)
    bn11r = bn11.reshape(1, -1)
    bn20r = bn20.reshape(1, -1)
    bn21r = bn21.reshape(1, -1)
    bur = bu.reshape(1, -1)

    u_red = pl.pallas_call(
        _ured_body,
        grid=(1,),
        in_specs=[_full((16, 4096)), _full((4096, 32)), _full((1, 32))],
        out_specs=_full((16, 32)),
        out_shape=jax.ShapeDtypeStruct((16, 32), f32),
    )(u, Wu, bur)

    H = E_PAD // 2
    ew = (u_red, W0r, W0c, W0u, w0e, be0r, We1, be1r, We2, be2r, W1c, W1e,
          bn10r, Wn11, bn11r)

    xr0, xc0 = _sc_gather(x16, rowp[:H], colp[:H])
    xr1, xc1 = _sc_gather(x16, rowp[H:], colp[H:])
    m0a, m1a, m2a, m3a = _edge_call(xr0, xc0, eap[:H], *ew)
    m0b, m1b, m2b, m3b = _edge_call(xr1, xc1, eap[H:], *ew)
    s0a, s1a, s2a, s3a, cnta = _sc_scatter(m0a, m1a, m2a, m3a, rowp[:H])
    s0b, s1b, s2b, s3b, cntb = _sc_scatter(m0b, m1b, m2b, m3b, rowp[H:])

    gn = N_PAD // BN
    out = pl.pallas_call(
        _node_body,
        grid=(gn,),
        in_specs=[
            pl.BlockSpec((BN, 16), lambda i: (i, 0)),
            pl.BlockSpec((BN, 16), lambda i: (i, 0)),
            pl.BlockSpec((BN, 16), lambda i: (i, 0)),
            pl.BlockSpec((BN, 16), lambda i: (i, 0)),
            pl.BlockSpec((BN, 16), lambda i: (i, 0)),
            pl.BlockSpec((BN, 16), lambda i: (i, 0)),
            pl.BlockSpec((BN, 16), lambda i: (i, 0)),
            pl.BlockSpec((BN, 16), lambda i: (i, 0)),
            pl.BlockSpec((BN, 16), lambda i: (i, 0)),
            pl.BlockSpec((BN, 1), lambda i: (i, 0)),
            pl.BlockSpec((BN, 1), lambda i: (i, 0)),
            pl.BlockSpec((BN, 1), lambda i: (i, 0)),
            _full((16, 32)),
            _full((16, 64)), _full((16, 64)), _full((16, 64)), _full((16, 64)),
            _full((16, 64)), _full((32, 64)),
            _full((1, 64)), _full((64, 1)), _full((1, 1)),
        ],
        out_specs=pl.BlockSpec((BN, 1), lambda i: (i, 0)),
        out_shape=jax.ShapeDtypeStruct((N_PAD, 1), f32),
    )(x16, s0a, s1a, s2a, s3a, s0b, s1b, s2b, s3b,
      cnta[:, None], cntb[:, None], batchp[:, None], u_red,
      W2x, W2a, W2b, W2c, W2d, W2u, bn20r, Wn21, bn21r)

    return out[:N_NODES, 0]


# speedup vs baseline: 1.1126x; 1.0825x over previous
import jax
import jax.numpy as jnp
from jax import lax
from jax.experimental import pallas as pl
from jax.experimental.pallas import tpu as pltpu
from jax.experimental.pallas import tpu_sc as plsc

N_NODES = 50000
N_GRAPHS = 16

E_PAD = 819200
N_PAD = 50176
BE = 2048
BN = 1024
DUMP = N_NODES

SC_K = 1600
NPT = N_PAD // 16
_SC_MESH = dict(core_axis_name="c", subcore_axis_name="s")
_SC_PARAMS = pltpu.CompilerParams(use_tc_tiling_on_sc=False)


def _ured_body(u_ref, wu_ref, bu_ref, out_ref):
    out_ref[...] = (
        jnp.dot(u_ref[...], wu_ref[...], preferred_element_type=jnp.float32)
        + bu_ref[...]
    )


def _edge_body(xr_ref, xc_ref, ured_ref, w0r_ref, w0c_ref, w0u_ref,
               be0_ref, we1_ref, be1_ref, we2_ref, be2_ref, w1c_ref,
               w1e_ref, bn10_ref, wn11_ref, bn11_ref,
               m0_ref, m1_ref, m2_ref, m3_ref):
    f32 = jnp.float32
    xr = xr_ref[...]
    xc = xc_ref[...]
    b = xr[:, 9:10]
    iota = lax.broadcasted_iota(jnp.int32, (1, N_GRAPHS), 1).astype(f32)
    oh = (b == iota).astype(f32)
    ub = jnp.dot(oh, ured_ref[...], preferred_element_type=f32)
    e0 = (jnp.dot(xr, w0r_ref[...], preferred_element_type=f32)
          + jnp.dot(xc, w0c_ref[...], preferred_element_type=f32)
          + jnp.dot(ub, w0u_ref[...], preferred_element_type=f32)
          + be0_ref[...])
    h = jnp.maximum(e0, 0.0)
    h = jnp.maximum(jnp.dot(h, we1_ref[...], preferred_element_type=f32)
                    + be1_ref[...], 0.0)
    eo = jnp.dot(h, we2_ref[...], preferred_element_type=f32) + be2_ref[...]
    m = jnp.maximum(jnp.dot(xc, w1c_ref[...], preferred_element_type=f32)
                    + jnp.dot(eo, w1e_ref[...], preferred_element_type=f32)
                    + bn10_ref[...], 0.0)
    m = jnp.maximum(jnp.dot(m, wn11_ref[...], preferred_element_type=f32)
                    + bn11_ref[...], 0.0)
    m0_ref[...] = m[:, 0:16]
    m1_ref[...] = m[:, 16:32]
    m2_ref[...] = m[:, 32:48]
    m3_ref[...] = m[:, 48:64]


def _node_body(x_ref, s0a_ref, s1a_ref, s2a_ref, s3a_ref,
               s0b_ref, s1b_ref, s2b_ref, s3b_ref, cnta_ref, cntb_ref,
               batch_ref, ured_ref, w2x_ref, w2a_ref, w2b_ref, w2c_ref,
               w2d_ref, w2u_ref, bn20_ref, wn21_ref, bn21_ref, out_ref):
    f32 = jnp.float32
    inv = 1.0 / jnp.maximum(cnta_ref[...] + cntb_ref[...], 1.0)
    b = batch_ref[...]
    oh = (b == lax.broadcasted_iota(jnp.int32, (1, N_GRAPHS), 1)).astype(f32)
    ub = jnp.dot(oh, ured_ref[...], preferred_element_type=f32)
    h2 = (jnp.dot(x_ref[...], w2x_ref[...], preferred_element_type=f32)
          + jnp.dot((s0a_ref[...] + s0b_ref[...]) * inv, w2a_ref[...],
                    preferred_element_type=f32)
          + jnp.dot((s1a_ref[...] + s1b_ref[...]) * inv, w2b_ref[...],
                    preferred_element_type=f32)
          + jnp.dot((s2a_ref[...] + s2b_ref[...]) * inv, w2c_ref[...],
                    preferred_element_type=f32)
          + jnp.dot((s3a_ref[...] + s3b_ref[...]) * inv, w2d_ref[...],
                    preferred_element_type=f32)
          + jnp.dot(ub, w2u_ref[...], preferred_element_type=f32)
          + bn20_ref[...])
    h2 = jnp.maximum(h2, 0.0)
    out_ref[...] = (jnp.dot(h2, wn21_ref[...], preferred_element_type=f32)
                    + bn21_ref[...])


def _full(shape):
    return pl.BlockSpec(shape, lambda i: (0,) * len(shape))


def _gather_body(x16_hbm, rowp_hbm, colp_hbm, eap_hbm, xr_hbm, xcp_hbm,
                 row_v, col_v, ea_v, xr_v, xc_v, sem1, sem2):
    c = lax.axis_index("c")
    s = lax.axis_index("s")
    wid = s * 2 + c
    epw = rowp_hbm.shape[0] // 32
    base_w = wid * epw
    lane10 = jnp.full((16,), 10, jnp.int32)

    @pl.loop(0, epw // SC_K)
    def _chunk(it):
        eb = base_w + it * SC_K
        pltpu.sync_copy(rowp_hbm.at[pl.ds(eb, SC_K)], row_v)
        pltpu.sync_copy(colp_hbm.at[pl.ds(eb, SC_K)], col_v)
        pltpu.sync_copy(eap_hbm.at[pl.ds(eb, SC_K)], ea_v)
        cp1 = pltpu.async_copy(x16_hbm.at[row_v], xr_v, sem1)
        cp2 = pltpu.async_copy(x16_hbm.at[col_v], xc_v, sem2)
        cp1.wait()
        cp2.wait()

        @pl.loop(0, SC_K // 16)
        def _ea(i):
            rows = lax.iota(jnp.int32, 16) + i * 16
            plsc.store_scatter(xr_v, [rows, lane10], ea_v[pl.ds(i * 16, 16)])

        pltpu.sync_copy(xr_v, xr_hbm.at[pl.ds(eb, SC_K)])
        pltpu.sync_copy(xc_v, xcp_hbm.at[pl.ds(eb, SC_K)])


def _sc_gather(x16, rowp, colp, eap):
    f32 = jnp.float32
    ne = rowp.shape[0]
    return pl.kernel(
        _gather_body,
        out_type=[
            jax.ShapeDtypeStruct((ne, 16), f32),
            jax.ShapeDtypeStruct((ne, 16), f32),
        ],
        mesh=plsc.VectorSubcoreMesh(**_SC_MESH),
        scratch_types=[
            pltpu.VMEM((SC_K,), jnp.int32),
            pltpu.VMEM((SC_K,), jnp.int32),
            pltpu.VMEM((SC_K,), f32),
            pltpu.VMEM((SC_K, 16), f32),
            pltpu.VMEM((SC_K, 16), f32),
            pltpu.SemaphoreType.DMA,
            pltpu.SemaphoreType.DMA,
        ],
        compiler_params=pltpu.CompilerParams(use_tc_tiling_on_sc=False,
                                             needs_layout_passes=False),
    )(x16, rowp, colp, eap)


def _scatter_body(m0_hbm, m1_hbm, m2_hbm, m3_hbm, rowp_hbm, z16_hbm, zcol_hbm,
                  ones_hbm, s0_hbm, s1_hbm, s2_hbm, s3_hbm, cnt_hbm,
                  row_v, m_v, ones_v, acc_sh, cnt_sh):
    c = lax.axis_index("c")
    s = lax.axis_index("s")
    r0 = s * NPT
    ept = rowp_hbm.shape[0] // 16
    base_t = s * ept

    for p in range(2):
        pltpu.sync_copy(z16_hbm, acc_sh.at[pl.ds(r0, NPT)])
        if p == 0:
            @pl.when(c == 0)
            def _():
                pltpu.sync_copy(zcol_hbm, cnt_sh.at[pl.ds(r0, NPT)])
                pltpu.sync_copy(ones_hbm, ones_v)

        plsc.subcore_barrier()

        @pl.loop(0, ept // SC_K)
        def _chunk(it):
            eb = base_t + it * SC_K
            pltpu.sync_copy(rowp_hbm.at[pl.ds(eb, SC_K)], row_v)

            @pl.when(c == 0)
            def _():
                pltpu.sync_copy((m0_hbm, m1_hbm)[p].at[pl.ds(eb, SC_K)], m_v)

            @pl.when(c == 1)
            def _():
                pltpu.sync_copy((m2_hbm, m3_hbm)[p].at[pl.ds(eb, SC_K)], m_v)

            pltpu.sync_copy(m_v, acc_sh.at[row_v], add=True)
            if p == 0:
                @pl.when(c == 0)
                def _():
                    pltpu.sync_copy(ones_v, cnt_sh.at[row_v], add=True)

        plsc.subcore_barrier()

        @pl.when(c == 0)
        def _():
            pltpu.sync_copy(acc_sh.at[pl.ds(r0, NPT)],
                            (s0_hbm, s1_hbm)[p].at[pl.ds(r0, NPT)])
            if p == 0:
                pltpu.sync_copy(cnt_sh.at[pl.ds(r0, NPT)],
                                cnt_hbm.at[pl.ds(r0, NPT)])

        @pl.when(c == 1)
        def _():
            pltpu.sync_copy(acc_sh.at[pl.ds(r0, NPT)],
                            (s2_hbm, s3_hbm)[p].at[pl.ds(r0, NPT)])


def _sc_scatter(m0, m1, m2, m3, rowp):
    f32 = jnp.float32
    z16 = jnp.zeros((NPT, 16), f32)
    zcol = jnp.zeros((NPT,), f32)
    ones = jnp.ones((SC_K,), f32)
    return pl.kernel(
        _scatter_body,
        out_type=[
            jax.ShapeDtypeStruct((N_PAD, 16), f32),
            jax.ShapeDtypeStruct((N_PAD, 16), f32),
            jax.ShapeDtypeStruct((N_PAD, 16), f32),
            jax.ShapeDtypeStruct((N_PAD, 16), f32),
            jax.ShapeDtypeStruct((N_PAD,), f32),
        ],
        mesh=plsc.VectorSubcoreMesh(**_SC_MESH),
        scratch_types=[
            pltpu.VMEM((SC_K,), jnp.int32),
            pltpu.VMEM((SC_K, 16), f32),
            pltpu.VMEM((SC_K,), f32),
            pltpu.VMEM_SHARED((N_PAD, 16), f32),
            pltpu.VMEM_SHARED((N_PAD,), f32),
        ],
        compiler_params=_SC_PARAMS,
    )(m0, m1, m2, m3, rowp, z16, zcol, ones)


def _edge_call(xr, xc, u_red, W0r, W0c, W0u, be0r, We1, be1r,
               We2, be2r, W1c, W1e, bn10r, Wn11, bn11r):
    f32 = jnp.float32
    ne = xr.shape[0]
    ge = ne // BE
    xp = pl.BlockSpec((BE, 16), lambda i: (i, 0))
    return pl.pallas_call(
        _edge_body,
        grid=(ge,),
        in_specs=[
            xp, xp,
            _full((16, 32)),
            _full((16, 64)), _full((16, 64)), _full((32, 64)),
            _full((1, 64)),
            _full((64, 64)), _full((1, 64)),
            _full((64, 64)), _full((1, 64)),
            _full((16, 64)), _full((64, 64)), _full((1, 64)),
            _full((64, 64)), _full((1, 64)),
        ],
        out_specs=[xp, xp, xp, xp],
        out_shape=[
            jax.ShapeDtypeStruct((ne, 16), f32),
            jax.ShapeDtypeStruct((ne, 16), f32),
            jax.ShapeDtypeStruct((ne, 16), f32),
            jax.ShapeDtypeStruct((ne, 16), f32),
        ],
    )(xr, xc, u_red, W0r, W0c, W0u, be0r, We1, be1r,
      We2, be2r, W1c, W1e, bn10r, Wn11, bn11r)


def kernel(x, edge_index, edge_attr, u, batch, Wu, bu, We0, be0, We1, be1,
           We2, be2, Wn10, bn10, Wn11, bn11, Wn20, bn20, Wn21, bn21):
    f32 = jnp.float32
    row = edge_index[0]
    col = edge_index[1]
    ne = row.shape[0]

    batchp = jnp.pad(batch, (0, N_PAD - N_NODES))
    x16 = jnp.pad(x, ((0, N_PAD - N_NODES), (0, 16 - x.shape[1])))
    x16 = x16.at[:, 9].set(batchp.astype(f32))
    rowp = jnp.concatenate([row, jnp.full((E_PAD - ne,), DUMP, jnp.int32)])
    colp = jnp.concatenate([col, jnp.zeros((E_PAD - ne,), jnp.int32)])
    eap = jnp.concatenate([edge_attr[:, 0], jnp.zeros((E_PAD - ne,), f32)])

    z64 = jnp.zeros((16, 64), f32)
    W0r = z64.at[:9].set(We0[0:9]).at[10].set(We0[18])
    W0c = z64.at[:9].set(We0[9:18])
    W0u = We0[19:51]
    W1c = z64.at[:9].set(Wn10[0:9])
    W1e = Wn10[9:73]
    W2x = z64.at[:9].set(Wn20[0:9])
    W2a = Wn20[9:25]
    W2b = Wn20[25:41]
    W2c = Wn20[41:57]
    W2d = Wn20[57:73]
    W2u = Wn20[73:105]
    be0r = be0.reshape(1, -1)
    be1r = be1.reshape(1, -1)
    be2r = be2.reshape(1, -1)
    bn10r = bn10.reshape(1, -1)
    bn11r = bn11.reshape(1, -1)
    bn20r = bn20.reshape(1, -1)
    bn21r = bn21.reshape(1, -1)
    bur = bu.reshape(1, -1)

    u_red = pl.pallas_call(
        _ured_body,
        grid=(1,),
        in_specs=[_full((16, 4096)), _full((4096, 32)), _full((1, 32))],
        out_specs=_full((16, 32)),
        out_shape=jax.ShapeDtypeStruct((16, 32), f32),
    )(u, Wu, bur)

    H = E_PAD // 2
    ew = (u_red, W0r, W0c, W0u, be0r, We1, be1r, We2, be2r, W1c, W1e,
          bn10r, Wn11, bn11r)

    xr0, xc0 = _sc_gather(x16, rowp[:H], colp[:H], eap[:H])
    xr1, xc1 = _sc_gather(x16, rowp[H:], colp[H:], eap[H:])
    m0a, m1a, m2a, m3a = _edge_call(xr0, xc0, *ew)
    m0b, m1b, m2b, m3b = _edge_call(xr1, xc1, *ew)
    s0a, s1a, s2a, s3a, cnta = _sc_scatter(m0a, m1a, m2a, m3a, rowp[:H])
    s0b, s1b, s2b, s3b, cntb = _sc_scatter(m0b, m1b, m2b, m3b, rowp[H:])

    gn = N_PAD // BN
    out = pl.pallas_call(
        _node_body,
        grid=(gn,),
        in_specs=[
            pl.BlockSpec((BN, 16), lambda i: (i, 0)),
            pl.BlockSpec((BN, 16), lambda i: (i, 0)),
            pl.BlockSpec((BN, 16), lambda i: (i, 0)),
            pl.BlockSpec((BN, 16), lambda i: (i, 0)),
            pl.BlockSpec((BN, 16), lambda i: (i, 0)),
            pl.BlockSpec((BN, 16), lambda i: (i, 0)),
            pl.BlockSpec((BN, 16), lambda i: (i, 0)),
            pl.BlockSpec((BN, 16), lambda i: (i, 0)),
            pl.BlockSpec((BN, 16), lambda i: (i, 0)),
            pl.BlockSpec((BN, 1), lambda i: (i, 0)),
            pl.BlockSpec((BN, 1), lambda i: (i, 0)),
            pl.BlockSpec((BN, 1), lambda i: (i, 0)),
            _full((16, 32)),
            _full((16, 64)), _full((16, 64)), _full((16, 64)), _full((16, 64)),
            _full((16, 64)), _full((32, 64)),
            _full((1, 64)), _full((64, 1)), _full((1, 1)),
        ],
        out_specs=pl.BlockSpec((BN, 1), lambda i: (i, 0)),
        out_shape=jax.ShapeDtypeStruct((N_PAD, 1), f32),
    )(x16, s0a, s1a, s2a, s3a, s0b, s1b, s2b, s3b,
      cnta[:N_PAD, None], cntb[:N_PAD, None], batchp[:, None], u_red,
      W2x, W2a, W2b, W2c, W2d, W2u, bn20r, Wn21, bn21r)

    return out[:N_NODES, 0]


# speedup vs baseline: 1.5764x; 1.4169x over previous
import jax
import jax.numpy as jnp
from jax import lax
from jax.experimental import pallas as pl
from jax.experimental.pallas import tpu as pltpu
from jax.experimental.pallas import tpu_sc as plsc

N_NODES = 50000
N_GRAPHS = 16

E_PAD = 819200
N_PAD = 50176
BE = 2048
BN = 1024
DUMP = N_NODES

SC_K = 1600
NPT = N_PAD // 16
_SC_MESH = dict(core_axis_name="c", subcore_axis_name="s")
_SC_PARAMS = pltpu.CompilerParams(use_tc_tiling_on_sc=False)


def _ured_body(u_ref, wu_ref, bu_ref, out_ref):
    out_ref[...] = (
        jnp.dot(u_ref[...], wu_ref[...], preferred_element_type=jnp.float32)
        + bu_ref[...]
    )


def _edge_body(xr_ref, xc_ref, ured_ref, w0r_ref, w0c_ref, w0u_ref,
               be0_ref, we1_ref, be1_ref, we2_ref, be2_ref, w1c_ref,
               w1e_ref, bn10_ref, wn11_ref, bn11_ref, mo_ref):
    f32 = jnp.float32
    xr = xr_ref[...]
    xc = xc_ref[...]
    b = xr[:, 9:10]
    iota = lax.broadcasted_iota(jnp.int32, (1, N_GRAPHS), 1).astype(f32)
    oh = (b == iota).astype(f32)
    ub = jnp.dot(oh, ured_ref[...], preferred_element_type=f32)
    e0 = (jnp.dot(xr, w0r_ref[...], preferred_element_type=f32)
          + jnp.dot(xc, w0c_ref[...], preferred_element_type=f32)
          + jnp.dot(ub, w0u_ref[...], preferred_element_type=f32)
          + be0_ref[...])
    h = jnp.maximum(e0, 0.0)
    h = jnp.maximum(jnp.dot(h, we1_ref[...], preferred_element_type=f32)
                    + be1_ref[...], 0.0)
    eo = jnp.dot(h, we2_ref[...], preferred_element_type=f32) + be2_ref[...]
    m = jnp.maximum(jnp.dot(xc, w1c_ref[...], preferred_element_type=f32)
                    + jnp.dot(eo, w1e_ref[...], preferred_element_type=f32)
                    + bn10_ref[...], 0.0)
    m = jnp.maximum(jnp.dot(m, wn11_ref[...], preferred_element_type=f32)
                    + bn11_ref[...], 0.0)
    mo_ref[...] = m


def _node_body(x_ref, sa_ref, sb_ref, cnta_ref, cntb_ref,
               batch_ref, ured_ref, w2x_ref, w2agg_ref,
               w2u_ref, bn20_ref, wn21_ref, bn21_ref, out_ref):
    f32 = jnp.float32
    inv = 1.0 / jnp.maximum(cnta_ref[...] + cntb_ref[...], 1.0)
    b = batch_ref[...]
    oh = (b == lax.broadcasted_iota(jnp.int32, (1, N_GRAPHS), 1)).astype(f32)
    ub = jnp.dot(oh, ured_ref[...], preferred_element_type=f32)
    agg = (sa_ref[...] + sb_ref[...]) * inv
    h2 = (jnp.dot(x_ref[...], w2x_ref[...], preferred_element_type=f32)
          + jnp.dot(agg, w2agg_ref[...], preferred_element_type=f32)
          + jnp.dot(ub, w2u_ref[...], preferred_element_type=f32)
          + bn20_ref[...])
    h2 = jnp.maximum(h2, 0.0)
    out_ref[...] = (jnp.dot(h2, wn21_ref[...], preferred_element_type=f32)
                    + bn21_ref[...])


def _full(shape):
    return pl.BlockSpec(shape, lambda i: (0,) * len(shape))


def _gather_body(x16_hbm, rowp_hbm, colp_hbm, eap_hbm, xr_hbm, xcp_hbm,
                 row_v, col_v, ea_v, xr_v, xc_v, sem1, sem2):
    c = lax.axis_index("c")
    s = lax.axis_index("s")
    wid = s * 2 + c
    epw = rowp_hbm.shape[0] // 32
    base_w = wid * epw
    lane10 = jnp.full((16,), 10, jnp.int32)

    @pl.loop(0, epw // SC_K)
    def _chunk(it):
        eb = base_w + it * SC_K
        pltpu.sync_copy(rowp_hbm.at[pl.ds(eb, SC_K)], row_v)
        pltpu.sync_copy(colp_hbm.at[pl.ds(eb, SC_K)], col_v)
        pltpu.sync_copy(eap_hbm.at[pl.ds(eb, SC_K)], ea_v)
        cp1 = pltpu.async_copy(x16_hbm.at[row_v], xr_v, sem1)
        cp2 = pltpu.async_copy(x16_hbm.at[col_v], xc_v, sem2)
        cp1.wait()
        cp2.wait()

        @pl.loop(0, SC_K // 16)
        def _ea(i):
            rows = lax.iota(jnp.int32, 16) + i * 16
            plsc.store_scatter(xr_v, [rows, lane10], ea_v[pl.ds(i * 16, 16)])

        pltpu.sync_copy(xr_v, xr_hbm.at[pl.ds(eb, SC_K)])
        pltpu.sync_copy(xc_v, xcp_hbm.at[pl.ds(eb, SC_K)])


def _sc_gather(x16, rowp, colp, eap):
    f32 = jnp.float32
    ne = rowp.shape[0]
    return pl.kernel(
        _gather_body,
        out_type=[
            jax.ShapeDtypeStruct((ne, 16), f32),
            jax.ShapeDtypeStruct((ne, 16), f32),
        ],
        mesh=plsc.VectorSubcoreMesh(**_SC_MESH),
        scratch_types=[
            pltpu.VMEM((SC_K,), jnp.int32),
            pltpu.VMEM((SC_K,), jnp.int32),
            pltpu.VMEM((SC_K,), f32),
            pltpu.VMEM((SC_K, 16), f32),
            pltpu.VMEM((SC_K, 16), f32),
            pltpu.SemaphoreType.DMA,
            pltpu.SemaphoreType.DMA,
        ],
        compiler_params=pltpu.CompilerParams(use_tc_tiling_on_sc=False,
                                             needs_layout_passes=False),
    )(x16, rowp, colp, eap)


def _scatter_body(m_hbm, rowp_hbm, z16_hbm, zcol_hbm, ones_hbm,
                  s_hbm, cnt_hbm, row_v, m_v, ones_v, acc_sh, cnt_sh):
    c = lax.axis_index("c")
    s = lax.axis_index("s")
    r0 = s * NPT
    ept = rowp_hbm.shape[0] // 16
    base_t = s * ept

    for p in range(2):
        col = c * 32 + p * 16
        pltpu.sync_copy(z16_hbm, acc_sh.at[pl.ds(r0, NPT)])
        if p == 0:
            @pl.when(c == 0)
            def _():
                pltpu.sync_copy(zcol_hbm, cnt_sh.at[pl.ds(r0, NPT)])
                pltpu.sync_copy(ones_hbm, ones_v)

        plsc.subcore_barrier()

        @pl.loop(0, ept // SC_K)
        def _chunk(it):
            eb = base_t + it * SC_K
            pltpu.sync_copy(rowp_hbm.at[pl.ds(eb, SC_K)], row_v)
            pltpu.sync_copy(m_hbm.at[pl.ds(eb, SC_K), pl.ds(col, 16)], m_v)
            pltpu.sync_copy(m_v, acc_sh.at[row_v], add=True)
            if p == 0:
                @pl.when(c == 0)
                def _():
                    pltpu.sync_copy(ones_v, cnt_sh.at[row_v], add=True)

        plsc.subcore_barrier()

        pltpu.sync_copy(acc_sh.at[pl.ds(r0, NPT)],
                        s_hbm.at[pl.ds(r0, NPT), pl.ds(col, 16)])
        if p == 0:
            @pl.when(c == 0)
            def _():
                pltpu.sync_copy(cnt_sh.at[pl.ds(r0, NPT)],
                                cnt_hbm.at[pl.ds(r0, NPT)])


def _sc_scatter(m, rowp):
    f32 = jnp.float32
    z16 = jnp.zeros((NPT, 16), f32)
    zcol = jnp.zeros((NPT,), f32)
    ones = jnp.ones((SC_K,), f32)
    return pl.kernel(
        _scatter_body,
        out_type=[
            jax.ShapeDtypeStruct((N_PAD, 64), f32),
            jax.ShapeDtypeStruct((N_PAD,), f32),
        ],
        mesh=plsc.VectorSubcoreMesh(**_SC_MESH),
        scratch_types=[
            pltpu.VMEM((SC_K,), jnp.int32),
            pltpu.VMEM((SC_K, 16), f32),
            pltpu.VMEM((SC_K,), f32),
            pltpu.VMEM_SHARED((N_PAD, 16), f32),
            pltpu.VMEM_SHARED((N_PAD,), f32),
        ],
        compiler_params=_SC_PARAMS,
    )(m, rowp, z16, zcol, ones)


def _edge_call(xr, xc, u_red, W0r, W0c, W0u, be0r, We1, be1r,
               We2, be2r, W1c, W1e, bn10r, Wn11, bn11r):
    f32 = jnp.float32
    ne = xr.shape[0]
    ge = ne // BE
    xp = pl.BlockSpec((BE, 16), lambda i: (i, 0))
    return pl.pallas_call(
        _edge_body,
        grid=(ge,),
        in_specs=[
            xp, xp,
            _full((16, 32)),
            _full((16, 64)), _full((16, 64)), _full((32, 64)),
            _full((1, 64)),
            _full((64, 64)), _full((1, 64)),
            _full((64, 64)), _full((1, 64)),
            _full((16, 64)), _full((64, 64)), _full((1, 64)),
            _full((64, 64)), _full((1, 64)),
        ],
        out_specs=pl.BlockSpec((BE, 64), lambda i: (i, 0)),
        out_shape=jax.ShapeDtypeStruct((ne, 64), f32),
    )(xr, xc, u_red, W0r, W0c, W0u, be0r, We1, be1r,
      We2, be2r, W1c, W1e, bn10r, Wn11, bn11r)


def kernel(x, edge_index, edge_attr, u, batch, Wu, bu, We0, be0, We1, be1,
           We2, be2, Wn10, bn10, Wn11, bn11, Wn20, bn20, Wn21, bn21):
    f32 = jnp.float32
    row = edge_index[0]
    col = edge_index[1]
    ne = row.shape[0]

    batchp = jnp.pad(batch, (0, N_PAD - N_NODES))
    x16 = jnp.pad(x, ((0, N_PAD - N_NODES), (0, 16 - x.shape[1])))
    x16 = x16.at[:, 9].set(batchp.astype(f32))
    rowp = jnp.concatenate([row, jnp.full((E_PAD - ne,), DUMP, jnp.int32)])
    colp = jnp.concatenate([col, jnp.zeros((E_PAD - ne,), jnp.int32)])
    eap = jnp.concatenate([edge_attr[:, 0], jnp.zeros((E_PAD - ne,), f32)])

    z64 = jnp.zeros((16, 64), f32)
    W0r = z64.at[:9].set(We0[0:9]).at[10].set(We0[18])
    W0c = z64.at[:9].set(We0[9:18])
    W0u = We0[19:51]
    W1c = z64.at[:9].set(Wn10[0:9])
    W1e = Wn10[9:73]
    W2x = z64.at[:9].set(Wn20[0:9])
    W2agg = Wn20[9:73]
    W2u = Wn20[73:105]
    be0r = be0.reshape(1, -1)
    be1r = be1.reshape(1, -1)
    be2r = be2.reshape(1, -1)
    bn10r = bn10.reshape(1, -1)
    bn11r = bn11.reshape(1, -1)
    bn20r = bn20.reshape(1, -1)
    bn21r = bn21.reshape(1, -1)
    bur = bu.reshape(1, -1)

    u_red = pl.pallas_call(
        _ured_body,
        grid=(1,),
        in_specs=[_full((16, 4096)), _full((4096, 32)), _full((1, 32))],
        out_specs=_full((16, 32)),
        out_shape=jax.ShapeDtypeStruct((16, 32), f32),
    )(u, Wu, bur)

    H = E_PAD // 2
    ew = (u_red, W0r, W0c, W0u, be0r, We1, be1r, We2, be2r, W1c, W1e,
          bn10r, Wn11, bn11r)

    xr0, xc0 = _sc_gather(x16, rowp[:H], colp[:H], eap[:H])
    xr1, xc1 = _sc_gather(x16, rowp[H:], colp[H:], eap[H:])
    ma = _edge_call(xr0, xc0, *ew)
    mb = _edge_call(xr1, xc1, *ew)
    sa, cnta = _sc_scatter(ma, rowp[:H])
    sb, cntb = _sc_scatter(mb, rowp[H:])

    gn = N_PAD // BN
    out = pl.pallas_call(
        _node_body,
        grid=(gn,),
        in_specs=[
            pl.BlockSpec((BN, 16), lambda i: (i, 0)),
            pl.BlockSpec((BN, 64), lambda i: (i, 0)),
            pl.BlockSpec((BN, 64), lambda i: (i, 0)),
            pl.BlockSpec((BN, 1), lambda i: (i, 0)),
            pl.BlockSpec((BN, 1), lambda i: (i, 0)),
            pl.BlockSpec((BN, 1), lambda i: (i, 0)),
            _full((16, 32)),
            _full((16, 64)), _full((64, 64)),
            _full((32, 64)),
            _full((1, 64)), _full((64, 1)), _full((1, 1)),
        ],
        out_specs=pl.BlockSpec((BN, 1), lambda i: (i, 0)),
        out_shape=jax.ShapeDtypeStruct((N_PAD, 1), f32),
    )(x16, sa, sb,
      cnta[:, None], cntb[:, None], batchp[:, None], u_red,
      W2x, W2agg, W2u, bn20r, Wn21, bn21r)

    return out[:N_NODES, 0]


# speedup vs baseline: 1.8829x; 1.1944x over previous
import jax
import jax.numpy as jnp
from jax import lax
from jax.experimental import pallas as pl
from jax.experimental.pallas import tpu as pltpu
from jax.experimental.pallas import tpu_sc as plsc

N_NODES = 50000
N_GRAPHS = 16

E_PAD = 819200
N_PAD = 50176
BE = 2048
BN = 1024
DUMP = N_NODES

SC_K = 1600
NPT = N_PAD // 16
_SC_MESH = dict(core_axis_name="c", subcore_axis_name="s")
_SC_PARAMS = pltpu.CompilerParams(use_tc_tiling_on_sc=False)


def _ured_body(u_ref, wu_ref, bu_ref, out_ref):
    out_ref[...] = (
        jnp.dot(u_ref[...], wu_ref[...], preferred_element_type=jnp.float32)
        + bu_ref[...]
    )


def _edge_body(x_ref, ured_ref, w0rc_ref, w0u_ref,
               be0_ref, we1_ref, be1_ref, we2_ref, be2_ref, w1c_ref,
               w1e_ref, bn10_ref, wn11_ref, bn11_ref, mo_ref):
    f32 = jnp.float32
    x = x_ref[...]
    b = x[:, 9:10]
    iota = lax.broadcasted_iota(jnp.int32, (1, N_GRAPHS), 1).astype(f32)
    oh = (b == iota).astype(f32)
    ub = jnp.dot(oh, ured_ref[...], preferred_element_type=f32)
    e0 = (jnp.dot(x, w0rc_ref[...], preferred_element_type=f32)
          + jnp.dot(ub, w0u_ref[...], preferred_element_type=f32)
          + be0_ref[...])
    h = jnp.maximum(e0, 0.0)
    h = jnp.maximum(jnp.dot(h, we1_ref[...], preferred_element_type=f32)
                    + be1_ref[...], 0.0)
    eo = jnp.dot(h, we2_ref[...], preferred_element_type=f32) + be2_ref[...]
    m = jnp.maximum(jnp.dot(x, w1c_ref[...], preferred_element_type=f32)
                    + jnp.dot(eo, w1e_ref[...], preferred_element_type=f32)
                    + bn10_ref[...], 0.0)
    m = jnp.maximum(jnp.dot(m, wn11_ref[...], preferred_element_type=f32)
                    + bn11_ref[...], 0.0)
    mo_ref[...] = m


def _node_body(x_ref, sa_ref, sb_ref, cnta_ref, cntb_ref,
               batch_ref, ured_ref, w2x_ref, w2agg_ref,
               w2u_ref, bn20_ref, wn21_ref, bn21_ref, out_ref):
    f32 = jnp.float32
    inv = 1.0 / jnp.maximum(cnta_ref[...] + cntb_ref[...], 1.0)
    b = batch_ref[...]
    oh = (b == lax.broadcasted_iota(jnp.int32, (1, N_GRAPHS), 1)).astype(f32)
    ub = jnp.dot(oh, ured_ref[...], preferred_element_type=f32)
    agg = (sa_ref[...] + sb_ref[...]) * inv
    h2 = (jnp.dot(x_ref[...], w2x_ref[...], preferred_element_type=f32)
          + jnp.dot(agg, w2agg_ref[...], preferred_element_type=f32)
          + jnp.dot(ub, w2u_ref[...], preferred_element_type=f32)
          + bn20_ref[...])
    h2 = jnp.maximum(h2, 0.0)
    out_ref[...] = (jnp.dot(h2, wn21_ref[...], preferred_element_type=f32)
                    + bn21_ref[...])


def _full(shape):
    return pl.BlockSpec(shape, lambda i: (0,) * len(shape))


def _gather_body(x16_hbm, rowp_hbm, colp_hbm, eap_hbm, xr_hbm,
                 row_v, col_v, ea_v, xr_v, xc_v, sem1, sem2):
    c = lax.axis_index("c")
    s = lax.axis_index("s")
    wid = s * 2 + c
    epw = rowp_hbm.shape[0] // 32
    base_w = wid * epw
    lane10 = jnp.full((16,), 10, jnp.int32)

    @pl.loop(0, epw // SC_K)
    def _chunk(it):
        eb = base_w + it * SC_K
        pltpu.sync_copy(rowp_hbm.at[pl.ds(eb, SC_K)], row_v)
        pltpu.sync_copy(colp_hbm.at[pl.ds(eb, SC_K)], col_v)
        pltpu.sync_copy(eap_hbm.at[pl.ds(eb, SC_K)], ea_v)
        cp1 = pltpu.async_copy(x16_hbm.at[row_v], xr_v, sem1)
        cp2 = pltpu.async_copy(x16_hbm.at[col_v], xc_v, sem2)
        cp1.wait()
        cp2.wait()

        @pl.loop(0, SC_K // 16)
        def _ea(i):
            rows = lax.iota(jnp.int32, 16) + i * 16
            plsc.store_scatter(xr_v, [rows, lane10], ea_v[pl.ds(i * 16, 16)])

        pltpu.sync_copy(xr_v, xr_hbm.at[pl.ds(eb, SC_K), pl.ds(0, 16)])
        pltpu.sync_copy(xc_v, xr_hbm.at[pl.ds(eb, SC_K), pl.ds(16, 16)])


def _sc_gather(x16, rowp, colp, eap):
    f32 = jnp.float32
    ne = rowp.shape[0]
    return pl.kernel(
        _gather_body,
        out_type=jax.ShapeDtypeStruct((ne, 32), f32),
        mesh=plsc.VectorSubcoreMesh(**_SC_MESH),
        scratch_types=[
            pltpu.VMEM((SC_K,), jnp.int32),
            pltpu.VMEM((SC_K,), jnp.int32),
            pltpu.VMEM((SC_K,), f32),
            pltpu.VMEM((SC_K, 16), f32),
            pltpu.VMEM((SC_K, 16), f32),
            pltpu.SemaphoreType.DMA,
            pltpu.SemaphoreType.DMA,
        ],
        compiler_params=pltpu.CompilerParams(use_tc_tiling_on_sc=False,
                                             needs_layout_passes=False),
    )(x16, rowp, colp, eap)


def _scatter_body(m_hbm, rowp_hbm, z16_hbm, zcol_hbm, ones_hbm,
                  s_hbm, cnt_hbm, row_v, m_v, ones_v, acc_sh, cnt_sh):
    c = lax.axis_index("c")
    s = lax.axis_index("s")
    r0 = s * NPT
    ept = rowp_hbm.shape[0] // 16
    base_t = s * ept

    for p in range(2):
        col = c * 32 + p * 16
        pltpu.sync_copy(z16_hbm, acc_sh.at[pl.ds(r0, NPT)])
        if p == 0:
            @pl.when(c == 0)
            def _():
                pltpu.sync_copy(zcol_hbm, cnt_sh.at[pl.ds(r0, NPT)])
                pltpu.sync_copy(ones_hbm, ones_v)

        plsc.subcore_barrier()

        @pl.loop(0, ept // SC_K)
        def _chunk(it):
            eb = base_t + it * SC_K
            pltpu.sync_copy(rowp_hbm.at[pl.ds(eb, SC_K)], row_v)
            pltpu.sync_copy(m_hbm.at[pl.ds(eb, SC_K), pl.ds(col, 16)], m_v)
            pltpu.sync_copy(m_v, acc_sh.at[row_v], add=True)
            if p == 0:
                @pl.when(c == 0)
                def _():
                    pltpu.sync_copy(ones_v, cnt_sh.at[row_v], add=True)

        plsc.subcore_barrier()

        pltpu.sync_copy(acc_sh.at[pl.ds(r0, NPT)],
                        s_hbm.at[pl.ds(r0, NPT), pl.ds(col, 16)])
        if p == 0:
            @pl.when(c == 0)
            def _():
                pltpu.sync_copy(cnt_sh.at[pl.ds(r0, NPT)],
                                cnt_hbm.at[pl.ds(r0, NPT)])


def _sc_scatter(m, rowp):
    f32 = jnp.float32
    z16 = jnp.zeros((NPT, 16), f32)
    zcol = jnp.zeros((NPT,), f32)
    ones = jnp.ones((SC_K,), f32)
    return pl.kernel(
        _scatter_body,
        out_type=[
            jax.ShapeDtypeStruct((N_PAD, 64), f32),
            jax.ShapeDtypeStruct((N_PAD,), f32),
        ],
        mesh=plsc.VectorSubcoreMesh(**_SC_MESH),
        scratch_types=[
            pltpu.VMEM((SC_K,), jnp.int32),
            pltpu.VMEM((SC_K, 16), f32),
            pltpu.VMEM((SC_K,), f32),
            pltpu.VMEM_SHARED((N_PAD, 16), f32),
            pltpu.VMEM_SHARED((N_PAD,), f32),
        ],
        compiler_params=_SC_PARAMS,
    )(m, rowp, z16, zcol, ones)


def _edge_call(xrc, u_red, W0rc, W0u, be0r, We1, be1r,
               We2, be2r, W1c32, W1e, bn10r, Wn11, bn11r):
    f32 = jnp.float32
    ne = xrc.shape[0]
    ge = ne // BE
    return pl.pallas_call(
        _edge_body,
        grid=(ge,),
        in_specs=[
            pl.BlockSpec((BE, 32), lambda i: (i, 0)),
            _full((16, 32)),
            _full((32, 64)), _full((32, 64)),
            _full((1, 64)),
            _full((64, 64)), _full((1, 64)),
            _full((64, 64)), _full((1, 64)),
            _full((32, 64)), _full((64, 64)), _full((1, 64)),
            _full((64, 64)), _full((1, 64)),
        ],
        out_specs=pl.BlockSpec((BE, 64), lambda i: (i, 0)),
        out_shape=jax.ShapeDtypeStruct((ne, 64), f32),
    )(xrc, u_red, W0rc, W0u, be0r, We1, be1r,
      We2, be2r, W1c32, W1e, bn10r, Wn11, bn11r)


def kernel(x, edge_index, edge_attr, u, batch, Wu, bu, We0, be0, We1, be1,
           We2, be2, Wn10, bn10, Wn11, bn11, Wn20, bn20, Wn21, bn21):
    f32 = jnp.float32
    row = edge_index[0]
    col = edge_index[1]
    ne = row.shape[0]

    batchp = jnp.pad(batch, (0, N_PAD - N_NODES))
    x16 = jnp.pad(x, ((0, N_PAD - N_NODES), (0, 16 - x.shape[1])))
    x16 = x16.at[:, 9].set(batchp.astype(f32))
    rowp = jnp.concatenate([row, jnp.full((E_PAD - ne,), DUMP, jnp.int32)])
    colp = jnp.concatenate([col, jnp.zeros((E_PAD - ne,), jnp.int32)])
    eap = jnp.concatenate([edge_attr[:, 0], jnp.zeros((E_PAD - ne,), f32)])

    z64 = jnp.zeros((16, 64), f32)
    z32 = jnp.zeros((32, 64), f32)
    W0rc = (z32.at[:9].set(We0[0:9]).at[10].set(We0[18])
            .at[16:25].set(We0[9:18]))
    W0u = We0[19:51]
    W1c32 = z32.at[16:25].set(Wn10[0:9])
    W1e = Wn10[9:73]
    W2x = z64.at[:9].set(Wn20[0:9])
    W2agg = Wn20[9:73]
    W2u = Wn20[73:105]
    be0r = be0.reshape(1, -1)
    be1r = be1.reshape(1, -1)
    be2r = be2.reshape(1, -1)
    bn10r = bn10.reshape(1, -1)
    bn11r = bn11.reshape(1, -1)
    bn20r = bn20.reshape(1, -1)
    bn21r = bn21.reshape(1, -1)
    bur = bu.reshape(1, -1)

    u_red = pl.pallas_call(
        _ured_body,
        grid=(1,),
        in_specs=[_full((16, 4096)), _full((4096, 32)), _full((1, 32))],
        out_specs=_full((16, 32)),
        out_shape=jax.ShapeDtypeStruct((16, 32), f32),
    )(u, Wu, bur)

    H = E_PAD // 2
    ew = (u_red, W0rc, W0u, be0r, We1, be1r, We2, be2r, W1c32, W1e,
          bn10r, Wn11, bn11r)

    xrc0 = _sc_gather(x16, rowp[:H], colp[:H], eap[:H])
    xrc1 = _sc_gather(x16, rowp[H:], colp[H:], eap[H:])
    ma = _edge_call(xrc0, *ew)
    mb = _edge_call(xrc1, *ew)
    sa, cnta = _sc_scatter(ma, rowp[:H])
    sb, cntb = _sc_scatter(mb, rowp[H:])

    gn = N_PAD // BN
    out = pl.pallas_call(
        _node_body,
        grid=(gn,),
        in_specs=[
            pl.BlockSpec((BN, 16), lambda i: (i, 0)),
            pl.BlockSpec((BN, 64), lambda i: (i, 0)),
            pl.BlockSpec((BN, 64), lambda i: (i, 0)),
            pl.BlockSpec((BN, 1), lambda i: (i, 0)),
            pl.BlockSpec((BN, 1), lambda i: (i, 0)),
            pl.BlockSpec((BN, 1), lambda i: (i, 0)),
            _full((16, 32)),
            _full((16, 64)), _full((64, 64)),
            _full((32, 64)),
            _full((1, 64)), _full((64, 1)), _full((1, 1)),
        ],
        out_specs=pl.BlockSpec((BN, 1), lambda i: (i, 0)),
        out_shape=jax.ShapeDtypeStruct((N_PAD, 1), f32),
    )(x16, sa, sb,
      cnta[:, None], cntb[:, None], batchp[:, None], u_red,
      W2x, W2agg, W2u, bn20r, Wn21, bn21r)

    return out[:N_NODES, 0]
